# Initial kernel scaffold; baseline (speedup 1.0000x reference)
import jax
import jax.numpy as jnp
from jax.experimental import pallas as pl


def kernel(x, edge_attr, z_c, true_coords, params, edge_index, u_idx, v_idx, w_idx):
    raise NotImplementedError("write your pallas kernel here")


# speedup vs baseline: 1.0896x; 1.0896x over previous
import functools

import jax
import jax.numpy as jnp
from jax import lax
from jax.experimental import pallas as pl
from jax.experimental.pallas import tpu as pltpu
from jax.experimental.pallas import tpu_sc as plsc

N = 50000
E = 800000
H = 128
NC, NS, LANES = 2, 16, 16
NW = NC * NS
EPAD = 819200
ER = EPAD // 128
TPW = EPAD // NW
MBR = TPW // 128 // 8
ACC_ROWS = 51200
SLICES = 8
SW = H // SLICES
STRIPE = ACC_ROWS // NS
NB = 2000
GRID = N // NB

_f32 = jnp.float32
_i32 = jnp.int32


def _elu(v):
    return jnp.where(v > 0.0, v, jnp.exp(v) - 1.0)


def _enc_body(x_ref, wn_ref, bn_ref, wm_ref, bm_ref, h_ref, *outs):
    h = _elu(jnp.dot(x_ref[...], wn_ref[...], preferred_element_type=_f32)
             + bn_ref[...])
    h_ref[...] = h
    hw = jnp.dot(h, wm_ref[...], preferred_element_type=_f32) + bm_ref[...]
    for k in range(SLICES):
        outs[k][...] = hw[:, SW * k:SW * (k + 1)]


def _encode(x, wn, bn, wm, bm):
    return pl.pallas_call(
        _enc_body,
        grid=(GRID,),
        in_specs=[
            pl.BlockSpec((NB, 4), lambda i: (i, 0)),
            pl.BlockSpec((4, H), lambda i: (0, 0)),
            pl.BlockSpec((1, H), lambda i: (0, 0)),
            pl.BlockSpec((H, H), lambda i: (0, 0)),
            pl.BlockSpec((1, H), lambda i: (0, 0)),
        ],
        out_specs=[pl.BlockSpec((NB, H), lambda i: (i, 0))]
        + [pl.BlockSpec((NB, SW), lambda i: (i, 0)) for _ in range(SLICES)],
        out_shape=[jax.ShapeDtypeStruct((N, H), _f32)]
        + [jax.ShapeDtypeStruct((N, SW), _f32) for _ in range(SLICES)],
    )(x, wn, bn, wm, bm)


def _acc_update(h_ref, agg_ref, wu_ref, bu_ref):
    acc = jnp.zeros((NB, H), _f32) + bu_ref[...]
    for k in range(SLICES):
        a = agg_ref[0, k] + agg_ref[1, k]
        acc = acc + jnp.dot(a, wu_ref[k], preferred_element_type=_f32)
    return _elu(h_ref[...] + acc)


def _upd_body(h_ref, agg_ref, wu_ref, bu_ref, wm_ref, bm_ref,
              h_out, *outs):
    h = _acc_update(h_ref, agg_ref, wu_ref, bu_ref)
    h_out[...] = h
    hw = jnp.dot(h, wm_ref[...], preferred_element_type=_f32) + bm_ref[...]
    for k in range(SLICES):
        outs[k][...] = hw[:, SW * k:SW * (k + 1)]


def _update(h, agg, wuS, bu, wm, bm):
    return pl.pallas_call(
        _upd_body,
        grid=(GRID,),
        in_specs=[
            pl.BlockSpec((NB, H), lambda i: (i, 0)),
            pl.BlockSpec((2, SLICES, NB, SW), lambda i: (0, 0, i, 0)),
            pl.BlockSpec((SLICES, SW, H), lambda i: (0, 0, 0)),
            pl.BlockSpec((1, H), lambda i: (0, 0)),
            pl.BlockSpec((H, H), lambda i: (0, 0)),
            pl.BlockSpec((1, H), lambda i: (0, 0)),
        ],
        out_specs=[pl.BlockSpec((NB, H), lambda i: (i, 0))]
        + [pl.BlockSpec((NB, SW), lambda i: (i, 0)) for _ in range(SLICES)],
        out_shape=[jax.ShapeDtypeStruct((N, H), _f32)]
        + [jax.ShapeDtypeStruct((N, SW), _f32) for _ in range(SLICES)],
    )(h, agg, wuS, bu, wm, bm)


def _fin_body(h_ref, agg_ref, wu_ref, bu_ref, wt1_ref, bt1_ref, wt2_ref,
              bt2_ref, h_out, topo_out):
    h = _acc_update(h_ref, agg_ref, wu_ref, bu_ref)
    h_out[...] = h
    t1 = _elu(jnp.dot(h, wt1_ref[...], preferred_element_type=_f32)
              + bt1_ref[...])
    topo_out[...] = (jnp.dot(t1, wt2_ref[...], preferred_element_type=_f32)
                     + bt2_ref[...])


def _final(h, agg, wuS, bu, wt1, bt1, wt2, bt2):
    return pl.pallas_call(
        _fin_body,
        grid=(GRID,),
        in_specs=[
            pl.BlockSpec((NB, H), lambda i: (i, 0)),
            pl.BlockSpec((2, SLICES, NB, SW), lambda i: (0, 0, i, 0)),
            pl.BlockSpec((SLICES, SW, H), lambda i: (0, 0, 0)),
            pl.BlockSpec((1, H), lambda i: (0, 0)),
            pl.BlockSpec((H, 64), lambda i: (0, 0)),
            pl.BlockSpec((1, 64), lambda i: (0, 0)),
            pl.BlockSpec((64, 1), lambda i: (0, 0)),
            pl.BlockSpec((1, 1), lambda i: (0, 0)),
        ],
        out_specs=[
            pl.BlockSpec((NB, H), lambda i: (i, 0)),
            pl.BlockSpec((NB, 1), lambda i: (i, 0)),
        ],
        out_shape=[
            jax.ShapeDtypeStruct((N, H), _f32),
            jax.ShapeDtypeStruct((N, 1), _f32),
        ],
    )(h, agg, wuS, bu, wt1, bt1, wt2, bt2)


def _sc_compute(rows, attr_v, w0, b0, j):
    def group_step(g, carry):
        av = attr_v[j, pl.ds(g * LANES, LANES)]
        for t in range(LANES):
            e = g * LANES + t
            a = av[t]
            e0 = _elu(a * w0 + b0)
            r0 = rows[j, e, pl.ds(0, SW)]
            rows[j, e, pl.ds(0, SW)] = _elu(r0 + e0)
        return carry

    lax.fori_loop(0, 8, group_step, 0)


def _sc_edge_body(*refs):
    ts = refs[:SLICES]
    (src_r, dst_r, attr_r, we_r, be_r, out,
     src_v, dst_v, attr_v, rows, wev, bev, zbuf, acc, gsem) = refs[SLICES:]
    c = lax.axis_index("c")
    s = lax.axis_index("s")
    wid = c * NS + s

    pltpu.sync_copy(we_r, wev)
    pltpu.sync_copy(be_r, bev)

    def zb(r, carry):
        zbuf[r, pl.ds(0, SW)] = jnp.zeros((SW,), _f32)
        return carry

    lax.fori_loop(0, 800, zb, 0)

    for k in range(SLICES):
        tk = ts[k]
        w0 = wev[pl.ds(SW * k, SW)]
        b0 = bev[pl.ds(SW * k, SW)]

        def zstripe(r, carry):
            pltpu.sync_copy(zbuf, acc.at[pl.ds((s * 4 + r) * 800, 800)])
            return carry

        lax.fori_loop(0, 4, zstripe, 0)
        plsc.subcore_barrier()

        def macro(it, carry):
            base = wid * (MBR * 8) + it * 8
            pltpu.sync_copy(src_r.at[pl.ds(base, 8)], src_v)
            pltpu.sync_copy(dst_r.at[pl.ds(base, 8)], dst_v)
            pltpu.sync_copy(attr_r.at[pl.ds(base, 8)], attr_v)

            def jstep(j, carry2):
                pltpu.async_copy(tk.at[src_v.at[j]], rows.at[j], gsem).wait()
                _sc_compute(rows, attr_v, w0, b0, j)
                pltpu.sync_copy(rows.at[j], acc.at[dst_v.at[j]], add=True)
                return carry2

            lax.fori_loop(0, 8, jstep, 0)
            return carry

        lax.fori_loop(0, MBR, macro, 0)
        plsc.subcore_barrier()
        pltpu.sync_copy(acc.at[pl.ds(s * STRIPE, STRIPE)],
                        out.at[c, k, pl.ds(s * STRIPE, STRIPE)])


_edge_sc = functools.partial(
    pl.kernel,
    out_type=jax.ShapeDtypeStruct((2, SLICES, ACC_ROWS, SW), _f32),
    mesh=plsc.VectorSubcoreMesh(core_axis_name="c", subcore_axis_name="s"),
    compiler_params=pltpu.CompilerParams(use_tc_tiling_on_sc=False),
    scratch_types=[
        pltpu.VMEM((8, 128), _i32),
        pltpu.VMEM((8, 128), _i32),
        pltpu.VMEM((8, 128), _f32),
        pltpu.VMEM((8, 128, SW), _f32),
        pltpu.VMEM((H,), _f32),
        pltpu.VMEM((H,), _f32),
        pltpu.VMEM((800, SW), _f32),
        pltpu.VMEM_SHARED((ACC_ROWS, SW), _f32),
        pltpu.SemaphoreType.DMA,
    ],
)(_sc_edge_body)


def _rows_body(uvw_ref, h_ref, out_ref):
    i = pl.program_id(0)
    r = uvw_ref[i] % 8
    out_ref[pl.ds(i, 1), :] = h_ref[pl.ds(r, 1), :]


def _head_rows(uvw, h):
    return pl.pallas_call(
        _rows_body,
        grid_spec=pltpu.PrefetchScalarGridSpec(
            num_scalar_prefetch=1,
            grid=(3,),
            in_specs=[pl.BlockSpec((8, H), lambda i, uvw: (uvw[i] // 8, 0))],
            out_specs=pl.BlockSpec((3, H), lambda i, uvw: (0, 0)),
        ),
        out_shape=jax.ShapeDtypeStruct((3, H), _f32),
    )(uvw, h)


def _head_body(r3, zc, tcrd, w1t, w1f, w1z, b1, w2, b2, wmu, bmu, wlv, blv,
               wd1z, wd1f, wd1zc, bd1, wd2, bd2, wd3, bd3, wp1f, wp1z, bp1,
               wp2, bp2, pbs, xp_ref, mu_ref, lv_ref):
    def dot(a, b):
        return jnp.dot(a, b, preferred_element_type=_f32)

    feat = (r3[0:1, :] + r3[1:2, :] + r3[2:3, :]) * (1.0 / 3.0)
    he = _elu(dot(tcrd[...], w1t[...]) + dot(feat, w1f[...])
              + dot(zc[...], w1z[...]) + b1[...])
    he = _elu(dot(he, w2[...]) + b2[...])
    mu = dot(he, wmu[...]) + bmu[...]
    lv = dot(he, wlv[...]) + blv[...]
    mu_ref[...] = mu
    lv_ref[...] = lv
    hd = _elu(dot(mu, wd1z[...]) + dot(feat, wd1f[...])
              + dot(zc[...], wd1zc[...]) + bd1[...])
    hd = _elu(dot(hd, wd2[...]) + bd2[...])
    base = dot(hd, wd3[...]) + bd3[...]
    pb = jnp.tanh(dot(_elu(dot(feat, wp1f[...]) + dot(zc[...], wp1z[...])
                           + bp1[...]), wp2[...]) + bp2[...])
    scale = jnp.clip(pbs[...], 0.0, 0.5)
    xp_ref[...] = base + pb * scale


def _head(r3, zc, tcrd, args):
    ins = [r3, zc, tcrd] + list(args)
    return pl.pallas_call(
        _head_body,
        in_specs=[pl.BlockSpec(a.shape, lambda: (0, 0)) for a in ins],
        out_specs=[
            pl.BlockSpec((1, 4), lambda: (0, 0)),
            pl.BlockSpec((1, 64), lambda: (0, 0)),
            pl.BlockSpec((1, 64), lambda: (0, 0)),
        ],
        out_shape=[
            jax.ShapeDtypeStruct((1, 4), _f32),
            jax.ShapeDtypeStruct((1, 64), _f32),
            jax.ShapeDtypeStruct((1, 64), _f32),
        ],
    )(*ins)


def kernel(x, edge_attr, z_c, true_coords, params, edge_index, u_idx, v_idx,
           w_idx):
    p = params
    src = edge_index[0].astype(_i32)
    dst = edge_index[1].astype(_i32)
    pad = EPAD - E
    srcp = jnp.concatenate([src, jnp.zeros((pad,), _i32)]).reshape(ER, 128)
    dstp = jnp.concatenate([dst, jnp.full((pad,), N, _i32)]).reshape(ER, 128)
    attrp = jnp.concatenate(
        [edge_attr[:, 0].astype(_f32), jnp.zeros((pad,), _f32)]
    ).reshape(ER, 128)
    we_r = p['We'][0]
    be_r = p['be']

    h, *ts = _encode(
        x, p['Wn'], p['bn'].reshape(1, H), p['Wm0'], p['bm0'].reshape(1, H))

    for l in range(4):
        agg = _edge_sc(*ts, srcp, dstp, attrp, we_r, be_r)
        wuS = p['Wu%d' % l].reshape(SLICES, SW, H)
        bu = p['bu%d' % l].reshape(1, H)
        if l < 3:
            h, *ts = _update(
                h, agg, wuS, bu,
                p['Wm%d' % (l + 1)], p['bm%d' % (l + 1)].reshape(1, H))
        else:
            h, topo2 = _final(
                h, agg, wuS, bu,
                p['Wt1'], p['bt1'].reshape(1, 64),
                p['Wt2'], p['bt2'].reshape(1, 1))

    uvw = jnp.stack([jnp.asarray(u_idx, _i32), jnp.asarray(v_idx, _i32),
                     jnp.asarray(w_idx, _i32)])
    r3 = _head_rows(uvw, h)

    we1 = p['We1']
    wd1 = p['Wd1']
    wp1 = p['Wp1']
    head_args = (
        we1[0:4], we1[4:132], we1[132:260], p['be1'].reshape(1, H),
        p['We2'], p['be2'].reshape(1, 64),
        p['Wmu'], p['bmu'].reshape(1, 64),
        p['Wlv'], p['blv'].reshape(1, 64),
        wd1[0:64], wd1[64:192], wd1[192:320], p['bd1'].reshape(1, H),
        p['Wd2'], p['bd2'].reshape(1, 64),
        p['Wd3'], p['bd3'].reshape(1, 4),
        wp1[0:128], wp1[128:256], p['bp1'].reshape(1, 64),
        p['Wp2'], p['bp2'].reshape(1, 4),
        p['pbs'].reshape(1, 1),
    )
    xp, mu, lv = _head(r3, z_c, true_coords, head_args)
    return topo2[:, 0], xp, mu, lv


# speedup vs baseline: 1.9197x; 1.7617x over previous
import functools

import jax
import jax.numpy as jnp
from jax import lax
from jax.experimental import pallas as pl
from jax.experimental.pallas import tpu as pltpu
from jax.experimental.pallas import tpu_sc as plsc

N = 50000
E = 800000
H = 128
NC, NS, LANES = 2, 16, 16
NW = NC * NS
EPAD = 819200
ER = EPAD // 128
TPW = EPAD // NW
MBR = TPW // 128 // 8
ACC_ROWS = 51200
SLICES = 8
SW = H // SLICES
STRIPE = ACC_ROWS // NS
NB = 2000
GRID = N // NB

_f32 = jnp.float32
_i32 = jnp.int32


def _elu(v):
    return jnp.where(v > 0.0, v, jnp.exp(v) - 1.0)


def _enc_body(x_ref, wn_ref, bn_ref, wm_ref, bm_ref, h_ref, *outs):
    h = _elu(jnp.dot(x_ref[...], wn_ref[...], preferred_element_type=_f32)
             + bn_ref[...])
    h_ref[...] = h
    hw = jnp.dot(h, wm_ref[...], preferred_element_type=_f32) + bm_ref[...]
    for k in range(SLICES):
        outs[k][...] = hw[:, SW * k:SW * (k + 1)]


def _encode(x, wn, bn, wm, bm):
    return pl.pallas_call(
        _enc_body,
        grid=(GRID,),
        in_specs=[
            pl.BlockSpec((NB, 4), lambda i: (i, 0)),
            pl.BlockSpec((4, H), lambda i: (0, 0)),
            pl.BlockSpec((1, H), lambda i: (0, 0)),
            pl.BlockSpec((H, H), lambda i: (0, 0)),
            pl.BlockSpec((1, H), lambda i: (0, 0)),
        ],
        out_specs=[pl.BlockSpec((NB, H), lambda i: (i, 0))]
        + [pl.BlockSpec((NB, SW), lambda i: (i, 0)) for _ in range(SLICES)],
        out_shape=[jax.ShapeDtypeStruct((N, H), _f32)]
        + [jax.ShapeDtypeStruct((N, SW), _f32) for _ in range(SLICES)],
    )(x, wn, bn, wm, bm)


def _acc_update(h_ref, agg_ref, wu_ref, bu_ref):
    acc = jnp.zeros((NB, H), _f32) + bu_ref[...]
    for k in range(SLICES):
        a = agg_ref[0, k] + agg_ref[1, k]
        acc = acc + jnp.dot(a, wu_ref[k], preferred_element_type=_f32)
    return _elu(h_ref[...] + acc)


def _upd_body(h_ref, agg_ref, wu_ref, bu_ref, wm_ref, bm_ref,
              h_out, *outs):
    h = _acc_update(h_ref, agg_ref, wu_ref, bu_ref)
    h_out[...] = h
    hw = jnp.dot(h, wm_ref[...], preferred_element_type=_f32) + bm_ref[...]
    for k in range(SLICES):
        outs[k][...] = hw[:, SW * k:SW * (k + 1)]


def _update(h, agg, wuS, bu, wm, bm):
    return pl.pallas_call(
        _upd_body,
        grid=(GRID,),
        in_specs=[
            pl.BlockSpec((NB, H), lambda i: (i, 0)),
            pl.BlockSpec((2, SLICES, NB, SW), lambda i: (0, 0, i, 0)),
            pl.BlockSpec((SLICES, SW, H), lambda i: (0, 0, 0)),
            pl.BlockSpec((1, H), lambda i: (0, 0)),
            pl.BlockSpec((H, H), lambda i: (0, 0)),
            pl.BlockSpec((1, H), lambda i: (0, 0)),
        ],
        out_specs=[pl.BlockSpec((NB, H), lambda i: (i, 0))]
        + [pl.BlockSpec((NB, SW), lambda i: (i, 0)) for _ in range(SLICES)],
        out_shape=[jax.ShapeDtypeStruct((N, H), _f32)]
        + [jax.ShapeDtypeStruct((N, SW), _f32) for _ in range(SLICES)],
    )(h, agg, wuS, bu, wm, bm)


def _fin_body(h_ref, agg_ref, wu_ref, bu_ref, wt1_ref, bt1_ref, wt2_ref,
              bt2_ref, h_out, topo_out):
    h = _acc_update(h_ref, agg_ref, wu_ref, bu_ref)
    h_out[...] = h
    t1 = _elu(jnp.dot(h, wt1_ref[...], preferred_element_type=_f32)
              + bt1_ref[...])
    topo_out[...] = (jnp.dot(t1, wt2_ref[...], preferred_element_type=_f32)
                     + bt2_ref[...])


def _final(h, agg, wuS, bu, wt1, bt1, wt2, bt2):
    return pl.pallas_call(
        _fin_body,
        grid=(GRID,),
        in_specs=[
            pl.BlockSpec((NB, H), lambda i: (i, 0)),
            pl.BlockSpec((2, SLICES, NB, SW), lambda i: (0, 0, i, 0)),
            pl.BlockSpec((SLICES, SW, H), lambda i: (0, 0, 0)),
            pl.BlockSpec((1, H), lambda i: (0, 0)),
            pl.BlockSpec((H, 64), lambda i: (0, 0)),
            pl.BlockSpec((1, 64), lambda i: (0, 0)),
            pl.BlockSpec((64, 1), lambda i: (0, 0)),
            pl.BlockSpec((1, 1), lambda i: (0, 0)),
        ],
        out_specs=[
            pl.BlockSpec((NB, H), lambda i: (i, 0)),
            pl.BlockSpec((NB, 1), lambda i: (i, 0)),
        ],
        out_shape=[
            jax.ShapeDtypeStruct((N, H), _f32),
            jax.ShapeDtypeStruct((N, 1), _f32),
        ],
    )(h, agg, wuS, bu, wt1, bt1, wt2, bt2)


def _sc_compute(rows, attr_v, w0, b0, p, j):
    def group_step(g, carry):
        av = attr_v[p, j, pl.ds(g * LANES, LANES)]
        for t in range(LANES):
            e = g * LANES + t
            a = av[t]
            e0 = _elu(a * w0 + b0)
            r0 = rows[p, j, e, pl.ds(0, SW)]
            rows[p, j, e, pl.ds(0, SW)] = _elu(r0 + e0)
        return carry

    lax.fori_loop(0, 8, group_step, 0)


def _sc_edge_body(*refs):
    ts = refs[:SLICES]
    (src_r, dst_r, attr_r, we_r, be_r, out,
     src_v, dst_v, attr_v, rows, wev, bev, zbuf, acc, gsem, ssem) = \
        refs[SLICES:]
    c = lax.axis_index("c")
    s = lax.axis_index("s")
    wid = c * NS + s

    pltpu.sync_copy(we_r, wev)
    pltpu.sync_copy(be_r, bev)

    def zb(r, carry):
        zbuf[r, pl.ds(0, SW)] = jnp.zeros((SW,), _f32)
        return carry

    lax.fori_loop(0, 800, zb, 0)

    def load_idx(it, ph):
        base = wid * (MBR * 8) + it * 8
        pltpu.sync_copy(src_r.at[pl.ds(base, 8)], src_v.at[ph])
        pltpu.sync_copy(dst_r.at[pl.ds(base, 8)], dst_v.at[ph])
        pltpu.sync_copy(attr_r.at[pl.ds(base, 8)], attr_v.at[ph])

    for k in range(SLICES):
        tk = ts[k]
        w0 = wev[pl.ds(SW * k, SW)]
        b0 = bev[pl.ds(SW * k, SW)]

        def fire_gathers(ph, carry=0):
            def fg(j, carry2):
                pltpu.async_copy(tk.at[src_v.at[ph, j]], rows.at[ph, j],
                                 gsem.at[ph])
                return carry2
            lax.fori_loop(0, 8, fg, 0)
            return carry

        def wait_gathers(ph):
            def wg(j, carry2):
                pltpu.make_async_copy(tk.at[src_v.at[ph, j]],
                                      rows.at[ph, j], gsem.at[ph]).wait()
                return carry2
            lax.fori_loop(0, 8, wg, 0)

        def fire_scatters(ph):
            def fs(j, carry2):
                pltpu.async_copy(rows.at[ph, j], acc.at[dst_v.at[ph, j]],
                                 ssem.at[ph], add=True)
                return carry2
            lax.fori_loop(0, 8, fs, 0)

        def drain_scatters(ph):
            def dsc(j, carry2):
                pltpu.make_async_copy(rows.at[ph, j],
                                      acc.at[dst_v.at[ph, j]],
                                      ssem.at[ph]).wait()
                return carry2
            lax.fori_loop(0, 8, dsc, 0)

        def zstripe(r, carry):
            pltpu.sync_copy(zbuf, acc.at[pl.ds((s * 4 + r) * 800, 800)])
            return carry

        lax.fori_loop(0, 4, zstripe, 0)
        plsc.subcore_barrier()

        load_idx(0, 0)
        fire_gathers(0)

        def macro(it, carry):
            p = lax.rem(it, 2)
            np_ = 1 - p

            @pl.when(it >= 1)
            def _():
                drain_scatters(np_)

            @pl.when(it + 1 < MBR)
            def _():
                load_idx(it + 1, np_)
                fire_gathers(np_)

            wait_gathers(p)

            def cstep(j, carry2):
                _sc_compute(rows, attr_v, w0, b0, p, j)
                return carry2

            lax.fori_loop(0, 8, cstep, 0)
            fire_scatters(p)
            return carry

        lax.fori_loop(0, MBR, macro, 0)
        drain_scatters((MBR - 1) % 2)
        plsc.subcore_barrier()
        pltpu.sync_copy(acc.at[pl.ds(s * STRIPE, STRIPE)],
                        out.at[c, k, pl.ds(s * STRIPE, STRIPE)])


_edge_sc = functools.partial(
    pl.kernel,
    out_type=jax.ShapeDtypeStruct((2, SLICES, ACC_ROWS, SW), _f32),
    mesh=plsc.VectorSubcoreMesh(core_axis_name="c", subcore_axis_name="s"),
    compiler_params=pltpu.CompilerParams(use_tc_tiling_on_sc=False),
    scratch_types=[
        pltpu.VMEM((2, 8, 128), _i32),
        pltpu.VMEM((2, 8, 128), _i32),
        pltpu.VMEM((2, 8, 128), _f32),
        pltpu.VMEM((2, 8, 128, SW), _f32),
        pltpu.VMEM((H,), _f32),
        pltpu.VMEM((H,), _f32),
        pltpu.VMEM((800, SW), _f32),
        pltpu.VMEM_SHARED((ACC_ROWS, SW), _f32),
        pltpu.SemaphoreType.DMA((2,)),
        pltpu.SemaphoreType.DMA((2,)),
    ],
)(_sc_edge_body)


def _rows_body(uvw_ref, h_ref, out_ref):
    i = pl.program_id(0)
    r = uvw_ref[i] % 8
    out_ref[pl.ds(i, 1), :] = h_ref[pl.ds(r, 1), :]


def _head_rows(uvw, h):
    return pl.pallas_call(
        _rows_body,
        grid_spec=pltpu.PrefetchScalarGridSpec(
            num_scalar_prefetch=1,
            grid=(3,),
            in_specs=[pl.BlockSpec((8, H), lambda i, uvw: (uvw[i] // 8, 0))],
            out_specs=pl.BlockSpec((3, H), lambda i, uvw: (0, 0)),
        ),
        out_shape=jax.ShapeDtypeStruct((3, H), _f32),
    )(uvw, h)


def _head_body(r3, zc, tcrd, w1t, w1f, w1z, b1, w2, b2, wmu, bmu, wlv, blv,
               wd1z, wd1f, wd1zc, bd1, wd2, bd2, wd3, bd3, wp1f, wp1z, bp1,
               wp2, bp2, pbs, xp_ref, mu_ref, lv_ref):
    def dot(a, b):
        return jnp.dot(a, b, preferred_element_type=_f32)

    feat = (r3[0:1, :] + r3[1:2, :] + r3[2:3, :]) * (1.0 / 3.0)
    he = _elu(dot(tcrd[...], w1t[...]) + dot(feat, w1f[...])
              + dot(zc[...], w1z[...]) + b1[...])
    he = _elu(dot(he, w2[...]) + b2[...])
    mu = dot(he, wmu[...]) + bmu[...]
    lv = dot(he, wlv[...]) + blv[...]
    mu_ref[...] = mu
    lv_ref[...] = lv
    hd = _elu(dot(mu, wd1z[...]) + dot(feat, wd1f[...])
              + dot(zc[...], wd1zc[...]) + bd1[...])
    hd = _elu(dot(hd, wd2[...]) + bd2[...])
    base = dot(hd, wd3[...]) + bd3[...]
    pb = jnp.tanh(dot(_elu(dot(feat, wp1f[...]) + dot(zc[...], wp1z[...])
                           + bp1[...]), wp2[...]) + bp2[...])
    scale = jnp.clip(pbs[...], 0.0, 0.5)
    xp_ref[...] = base + pb * scale


def _head(r3, zc, tcrd, args):
    ins = [r3, zc, tcrd] + list(args)
    return pl.pallas_call(
        _head_body,
        in_specs=[pl.BlockSpec(a.shape, lambda: (0, 0)) for a in ins],
        out_specs=[
            pl.BlockSpec((1, 4), lambda: (0, 0)),
            pl.BlockSpec((1, 64), lambda: (0, 0)),
            pl.BlockSpec((1, 64), lambda: (0, 0)),
        ],
        out_shape=[
            jax.ShapeDtypeStruct((1, 4), _f32),
            jax.ShapeDtypeStruct((1, 64), _f32),
            jax.ShapeDtypeStruct((1, 64), _f32),
        ],
    )(*ins)


def kernel(x, edge_attr, z_c, true_coords, params, edge_index, u_idx, v_idx,
           w_idx):
    p = params
    src = edge_index[0].astype(_i32)
    dst = edge_index[1].astype(_i32)
    pad = EPAD - E
    srcp = jnp.concatenate([src, jnp.zeros((pad,), _i32)]).reshape(ER, 128)
    dstp = jnp.concatenate([dst, jnp.full((pad,), N, _i32)]).reshape(ER, 128)
    attrp = jnp.concatenate(
        [edge_attr[:, 0].astype(_f32), jnp.zeros((pad,), _f32)]
    ).reshape(ER, 128)
    we_r = p['We'][0]
    be_r = p['be']

    h, *ts = _encode(
        x, p['Wn'], p['bn'].reshape(1, H), p['Wm0'], p['bm0'].reshape(1, H))

    for l in range(4):
        agg = _edge_sc(*ts, srcp, dstp, attrp, we_r, be_r)
        wuS = p['Wu%d' % l].reshape(SLICES, SW, H)
        bu = p['bu%d' % l].reshape(1, H)
        if l < 3:
            h, *ts = _update(
                h, agg, wuS, bu,
                p['Wm%d' % (l + 1)], p['bm%d' % (l + 1)].reshape(1, H))
        else:
            h, topo2 = _final(
                h, agg, wuS, bu,
                p['Wt1'], p['bt1'].reshape(1, 64),
                p['Wt2'], p['bt2'].reshape(1, 1))

    uvw = jnp.stack([jnp.asarray(u_idx, _i32), jnp.asarray(v_idx, _i32),
                     jnp.asarray(w_idx, _i32)])
    r3 = _head_rows(uvw, h)

    we1 = p['We1']
    wd1 = p['Wd1']
    wp1 = p['Wp1']
    head_args = (
        we1[0:4], we1[4:132], we1[132:260], p['be1'].reshape(1, H),
        p['We2'], p['be2'].reshape(1, 64),
        p['Wmu'], p['bmu'].reshape(1, 64),
        p['Wlv'], p['blv'].reshape(1, 64),
        wd1[0:64], wd1[64:192], wd1[192:320], p['bd1'].reshape(1, H),
        p['Wd2'], p['bd2'].reshape(1, 64),
        p['Wd3'], p['bd3'].reshape(1, 4),
        wp1[0:128], wp1[128:256], p['bp1'].reshape(1, 64),
        p['Wp2'], p['bp2'].reshape(1, 4),
        p['pbs'].reshape(1, 1),
    )
    xp, mu, lv = _head(r3, z_c, true_coords, head_args)
    return topo2[:, 0], xp, mu, lv


# speedup vs baseline: 2.0432x; 1.0644x over previous
import functools

import jax
import jax.numpy as jnp
from jax import lax
from jax.experimental import pallas as pl
from jax.experimental.pallas import tpu as pltpu
from jax.experimental.pallas import tpu_sc as plsc

N = 50000
E = 800000
H = 128
NC, NS, LANES = 2, 16, 16
NW = NC * NS
EPAD = 819200
ER = EPAD // 128
TPW = EPAD // NW
MBR = TPW // 128 // 8
ACC_ROWS = 51200
SLICES = 8
SW = H // SLICES
STRIPE = ACC_ROWS // NS
NB = 2000
GRID = N // NB

_f32 = jnp.float32
_i32 = jnp.int32


def _elu(v):
    return jnp.where(v > 0.0, v, jnp.exp(v) - 1.0)


def _enc_body(x_ref, wn_ref, bn_ref, wm_ref, bm_ref, h_ref, *outs):
    h = _elu(jnp.dot(x_ref[...], wn_ref[...], preferred_element_type=_f32)
             + bn_ref[...])
    h_ref[...] = h
    hw = jnp.dot(h, wm_ref[...], preferred_element_type=_f32) + bm_ref[...]
    for k in range(SLICES):
        outs[k][...] = hw[:, SW * k:SW * (k + 1)]


def _encode(x, wn, bn, wm, bm):
    return pl.pallas_call(
        _enc_body,
        grid=(GRID,),
        in_specs=[
            pl.BlockSpec((NB, 4), lambda i: (i, 0)),
            pl.BlockSpec((4, H), lambda i: (0, 0)),
            pl.BlockSpec((1, H), lambda i: (0, 0)),
            pl.BlockSpec((H, H), lambda i: (0, 0)),
            pl.BlockSpec((1, H), lambda i: (0, 0)),
        ],
        out_specs=[pl.BlockSpec((NB, H), lambda i: (i, 0))]
        + [pl.BlockSpec((NB, SW), lambda i: (i, 0)) for _ in range(SLICES)],
        out_shape=[jax.ShapeDtypeStruct((N, H), _f32)]
        + [jax.ShapeDtypeStruct((N, SW), _f32) for _ in range(SLICES)],
    )(x, wn, bn, wm, bm)


def _acc_update(h_ref, agg_ref, wu_ref, bu_ref):
    a = jnp.concatenate(
        [agg_ref[0, k] + agg_ref[1, k] for k in range(SLICES)], axis=-1)
    acc = jnp.dot(a, wu_ref[...], preferred_element_type=_f32) + bu_ref[...]
    return _elu(h_ref[...] + acc)


def _upd_body(h_ref, agg_ref, wu_ref, bu_ref, wm_ref, bm_ref,
              h_out, *outs):
    h = _acc_update(h_ref, agg_ref, wu_ref, bu_ref)
    h_out[...] = h
    hw = jnp.dot(h, wm_ref[...], preferred_element_type=_f32) + bm_ref[...]
    for k in range(SLICES):
        outs[k][...] = hw[:, SW * k:SW * (k + 1)]


def _update(h, agg, wuS, bu, wm, bm):
    return pl.pallas_call(
        _upd_body,
        grid=(GRID,),
        in_specs=[
            pl.BlockSpec((NB, H), lambda i: (i, 0)),
            pl.BlockSpec((2, SLICES, NB, SW), lambda i: (0, 0, i, 0)),
            pl.BlockSpec((H, H), lambda i: (0, 0)),
            pl.BlockSpec((1, H), lambda i: (0, 0)),
            pl.BlockSpec((H, H), lambda i: (0, 0)),
            pl.BlockSpec((1, H), lambda i: (0, 0)),
        ],
        out_specs=[pl.BlockSpec((NB, H), lambda i: (i, 0))]
        + [pl.BlockSpec((NB, SW), lambda i: (i, 0)) for _ in range(SLICES)],
        out_shape=[jax.ShapeDtypeStruct((N, H), _f32)]
        + [jax.ShapeDtypeStruct((N, SW), _f32) for _ in range(SLICES)],
    )(h, agg, wuS, bu, wm, bm)


def _fin_body(h_ref, agg_ref, wu_ref, bu_ref, wt1_ref, bt1_ref, wt2_ref,
              bt2_ref, h_out, topo_out):
    h = _acc_update(h_ref, agg_ref, wu_ref, bu_ref)
    h_out[...] = h
    t1 = _elu(jnp.dot(h, wt1_ref[...], preferred_element_type=_f32)
              + bt1_ref[...])
    topo_out[...] = (jnp.dot(t1, wt2_ref[...], preferred_element_type=_f32)
                     + bt2_ref[...])


def _final(h, agg, wuS, bu, wt1, bt1, wt2, bt2):
    return pl.pallas_call(
        _fin_body,
        grid=(GRID,),
        in_specs=[
            pl.BlockSpec((NB, H), lambda i: (i, 0)),
            pl.BlockSpec((2, SLICES, NB, SW), lambda i: (0, 0, i, 0)),
            pl.BlockSpec((H, H), lambda i: (0, 0)),
            pl.BlockSpec((1, H), lambda i: (0, 0)),
            pl.BlockSpec((H, 64), lambda i: (0, 0)),
            pl.BlockSpec((1, 64), lambda i: (0, 0)),
            pl.BlockSpec((64, 1), lambda i: (0, 0)),
            pl.BlockSpec((1, 1), lambda i: (0, 0)),
        ],
        out_specs=[
            pl.BlockSpec((NB, H), lambda i: (i, 0)),
            pl.BlockSpec((NB, 1), lambda i: (i, 0)),
        ],
        out_shape=[
            jax.ShapeDtypeStruct((N, H), _f32),
            jax.ShapeDtypeStruct((N, 1), _f32),
        ],
    )(h, agg, wuS, bu, wt1, bt1, wt2, bt2)


def _sc_compute(rows, attr_v, w0, b0, ph, j):
    def group_step(g, carry):
        av = attr_v[ph, j, pl.ds(g * LANES, LANES)]
        for t in range(LANES):
            e = g * LANES + t
            a = av[t]
            e0 = _elu(a * w0 + b0)
            r0 = rows[ph, j, e, pl.ds(0, SW)]
            rows[ph, j, e, pl.ds(0, SW)] = _elu(r0 + e0)
        return carry

    lax.fori_loop(0, 8, group_step, 0)


def _sc_edge_body(*refs):
    ts = refs[:SLICES]
    (src_r, dst_r, attr_r, we_r, be_r, out,
     src_v, dst_v, attr_v, rows, wev, bev, zbuf, acc, isem, gsem, ssem) = \
        refs[SLICES:]
    c = lax.axis_index("c")
    s = lax.axis_index("s")
    wid = c * NS + s
    tbase = wid * (MBR * 8)

    pltpu.sync_copy(we_r, wev)
    pltpu.sync_copy(be_r, bev)

    def zb(r, carry):
        zbuf[r, pl.ds(0, SW)] = jnp.zeros((SW,), _f32)
        return carry

    lax.fori_loop(0, 200, zb, 0)

    def load_idx_sync(it, ph):
        pltpu.sync_copy(src_r.at[pl.ds(tbase + it * 8, 8)], src_v.at[ph])
        pltpu.sync_copy(dst_r.at[pl.ds(tbase + it * 8, 8)], dst_v.at[ph])
        pltpu.sync_copy(attr_r.at[pl.ds(tbase + it * 8, 8)], attr_v.at[ph])

    def fire_idx(it, ph):
        pltpu.async_copy(src_r.at[pl.ds(tbase + it * 8, 8)], src_v.at[ph],
                         isem.at[ph])
        pltpu.async_copy(dst_r.at[pl.ds(tbase + it * 8, 8)], dst_v.at[ph],
                         isem.at[ph])
        pltpu.async_copy(attr_r.at[pl.ds(tbase + it * 8, 8)], attr_v.at[ph],
                         isem.at[ph])

    def wait_idx(it, ph):
        pltpu.make_async_copy(src_r.at[pl.ds(tbase + it * 8, 8)],
                              src_v.at[ph], isem.at[ph]).wait()
        pltpu.make_async_copy(dst_r.at[pl.ds(tbase + it * 8, 8)],
                              dst_v.at[ph], isem.at[ph]).wait()
        pltpu.make_async_copy(attr_r.at[pl.ds(tbase + it * 8, 8)],
                              attr_v.at[ph], isem.at[ph]).wait()

    for k in range(SLICES):
        tk = ts[k]
        w0 = wev[pl.ds(SW * k, SW)]
        b0 = bev[pl.ds(SW * k, SW)]

        def fire_gathers(ph, carry=0):
            def fg(j, carry2):
                pltpu.async_copy(tk.at[src_v.at[ph, j]], rows.at[ph, j],
                                 gsem.at[ph])
                return carry2
            lax.fori_loop(0, 8, fg, 0)
            return carry

        def wait_gathers(ph):
            def wg(j, carry2):
                pltpu.make_async_copy(tk.at[src_v.at[ph, j]],
                                      rows.at[ph, j], gsem.at[ph]).wait()
                return carry2
            lax.fori_loop(0, 8, wg, 0)

        def fire_scatters(ph):
            def fs(j, carry2):
                pltpu.async_copy(rows.at[ph, j], acc.at[dst_v.at[ph, j]],
                                 ssem.at[ph], add=True)
                return carry2
            lax.fori_loop(0, 8, fs, 0)

        def drain_scatters(ph):
            def dsc(j, carry2):
                pltpu.make_async_copy(rows.at[ph, j],
                                      acc.at[dst_v.at[ph, j]],
                                      ssem.at[ph]).wait()
                return carry2
            lax.fori_loop(0, 8, dsc, 0)

        def zstripe(r, carry):
            pltpu.sync_copy(zbuf, acc.at[pl.ds((s * 16 + r) * 200, 200)])
            return carry

        lax.fori_loop(0, 16, zstripe, 0)
        plsc.subcore_barrier()

        load_idx_sync(0, 0)
        fire_gathers(0)
        fire_idx(1, 1)

        def macro(it, carry):
            p = lax.rem(it, 3)
            pn = lax.rem(it + 1, 3)
            pnn = lax.rem(it + 2, 3)

            @pl.when(it + 1 < MBR)
            def _():
                wait_idx(it + 1, pn)
                fire_gathers(pn)

            @pl.when(it >= 1)
            def _():
                drain_scatters(pnn)

            @pl.when(it + 2 < MBR)
            def _():
                fire_idx(it + 2, pnn)

            wait_gathers(p)

            def cstep(j, carry2):
                _sc_compute(rows, attr_v, w0, b0, p, j)
                return carry2

            lax.fori_loop(0, 8, cstep, 0)
            fire_scatters(p)
            return carry

        lax.fori_loop(0, MBR, macro, 0)
        drain_scatters((MBR - 1) % 3)
        plsc.subcore_barrier()
        pltpu.sync_copy(acc.at[pl.ds(s * STRIPE, STRIPE)],
                        out.at[c, k, pl.ds(s * STRIPE, STRIPE)])


_edge_sc = functools.partial(
    pl.kernel,
    out_type=jax.ShapeDtypeStruct((2, SLICES, ACC_ROWS, SW), _f32),
    mesh=plsc.VectorSubcoreMesh(core_axis_name="c", subcore_axis_name="s"),
    compiler_params=pltpu.CompilerParams(use_tc_tiling_on_sc=False),
    scratch_types=[
        pltpu.VMEM((3, 8, 128), _i32),
        pltpu.VMEM((3, 8, 128), _i32),
        pltpu.VMEM((3, 8, 128), _f32),
        pltpu.VMEM((3, 8, 128, SW), _f32),
        pltpu.VMEM((H,), _f32),
        pltpu.VMEM((H,), _f32),
        pltpu.VMEM((200, SW), _f32),
        pltpu.VMEM_SHARED((ACC_ROWS, SW), _f32),
        pltpu.SemaphoreType.DMA((3,)),
        pltpu.SemaphoreType.DMA((3,)),
        pltpu.SemaphoreType.DMA((3,)),
    ],
)(_sc_edge_body)


def _rows_body(uvw_ref, h_ref, out_ref):
    i = pl.program_id(0)
    r = uvw_ref[i] % 8
    out_ref[pl.ds(i, 1), :] = h_ref[pl.ds(r, 1), :]


def _head_rows(uvw, h):
    return pl.pallas_call(
        _rows_body,
        grid_spec=pltpu.PrefetchScalarGridSpec(
            num_scalar_prefetch=1,
            grid=(3,),
            in_specs=[pl.BlockSpec((8, H), lambda i, uvw: (uvw[i] // 8, 0))],
            out_specs=pl.BlockSpec((3, H), lambda i, uvw: (0, 0)),
        ),
        out_shape=jax.ShapeDtypeStruct((3, H), _f32),
    )(uvw, h)


def _head_body(r3, zc, tcrd, w1t, w1f, w1z, b1, w2, b2, wmu, bmu, wlv, blv,
               wd1z, wd1f, wd1zc, bd1, wd2, bd2, wd3, bd3, wp1f, wp1z, bp1,
               wp2, bp2, pbs, xp_ref, mu_ref, lv_ref):
    def dot(a, b):
        return jnp.dot(a, b, preferred_element_type=_f32)

    feat = (r3[0:1, :] + r3[1:2, :] + r3[2:3, :]) * (1.0 / 3.0)
    he = _elu(dot(tcrd[...], w1t[...]) + dot(feat, w1f[...])
              + dot(zc[...], w1z[...]) + b1[...])
    he = _elu(dot(he, w2[...]) + b2[...])
    mu = dot(he, wmu[...]) + bmu[...]
    lv = dot(he, wlv[...]) + blv[...]
    mu_ref[...] = mu
    lv_ref[...] = lv
    hd = _elu(dot(mu, wd1z[...]) + dot(feat, wd1f[...])
              + dot(zc[...], wd1zc[...]) + bd1[...])
    hd = _elu(dot(hd, wd2[...]) + bd2[...])
    base = dot(hd, wd3[...]) + bd3[...]
    pb = jnp.tanh(dot(_elu(dot(feat, wp1f[...]) + dot(zc[...], wp1z[...])
                           + bp1[...]), wp2[...]) + bp2[...])
    scale = jnp.clip(pbs[...], 0.0, 0.5)
    xp_ref[...] = base + pb * scale


def _head(r3, zc, tcrd, args):
    ins = [r3, zc, tcrd] + list(args)
    return pl.pallas_call(
        _head_body,
        in_specs=[pl.BlockSpec(a.shape, lambda: (0, 0)) for a in ins],
        out_specs=[
            pl.BlockSpec((1, 4), lambda: (0, 0)),
            pl.BlockSpec((1, 64), lambda: (0, 0)),
            pl.BlockSpec((1, 64), lambda: (0, 0)),
        ],
        out_shape=[
            jax.ShapeDtypeStruct((1, 4), _f32),
            jax.ShapeDtypeStruct((1, 64), _f32),
            jax.ShapeDtypeStruct((1, 64), _f32),
        ],
    )(*ins)


def kernel(x, edge_attr, z_c, true_coords, params, edge_index, u_idx, v_idx,
           w_idx):
    p = params
    src = edge_index[0].astype(_i32)
    dst = edge_index[1].astype(_i32)
    pad = EPAD - E
    srcp = jnp.concatenate([src, jnp.zeros((pad,), _i32)]).reshape(ER, 128)
    dstp = jnp.concatenate([dst, jnp.full((pad,), N, _i32)]).reshape(ER, 128)
    attrp = jnp.concatenate(
        [edge_attr[:, 0].astype(_f32), jnp.zeros((pad,), _f32)]
    ).reshape(ER, 128)
    we_r = p['We'][0]
    be_r = p['be']

    h, *ts = _encode(
        x, p['Wn'], p['bn'].reshape(1, H), p['Wm0'], p['bm0'].reshape(1, H))

    for l in range(4):
        agg = _edge_sc(*ts, srcp, dstp, attrp, we_r, be_r)
        wuS = p['Wu%d' % l]
        bu = p['bu%d' % l].reshape(1, H)
        if l < 3:
            h, *ts = _update(
                h, agg, wuS, bu,
                p['Wm%d' % (l + 1)], p['bm%d' % (l + 1)].reshape(1, H))
        else:
            h, topo2 = _final(
                h, agg, wuS, bu,
                p['Wt1'], p['bt1'].reshape(1, 64),
                p['Wt2'], p['bt2'].reshape(1, 1))

    uvw = jnp.stack([jnp.asarray(u_idx, _i32), jnp.asarray(v_idx, _i32),
                     jnp.asarray(w_idx, _i32)])
    r3 = _head_rows(uvw, h)

    we1 = p['We1']
    wd1 = p['Wd1']
    wp1 = p['Wp1']
    head_args = (
        we1[0:4], we1[4:132], we1[132:260], p['be1'].reshape(1, H),
        p['We2'], p['be2'].reshape(1, 64),
        p['Wmu'], p['bmu'].reshape(1, 64),
        p['Wlv'], p['blv'].reshape(1, 64),
        wd1[0:64], wd1[64:192], wd1[192:320], p['bd1'].reshape(1, H),
        p['Wd2'], p['bd2'].reshape(1, 64),
        p['Wd3'], p['bd3'].reshape(1, 4),
        wp1[0:128], wp1[128:256], p['bp1'].reshape(1, 64),
        p['Wp2'], p['bp2'].reshape(1, 4),
        p['pbs'].reshape(1, 1),
    )
    xp, mu, lv = _head(r3, z_c, true_coords, head_args)
    return topo2[:, 0], xp, mu, lv


# speedup vs baseline: 2.0478x; 1.0023x over previous
import functools

import jax
import jax.numpy as jnp
from jax import lax
from jax.experimental import pallas as pl
from jax.experimental.pallas import tpu as pltpu
from jax.experimental.pallas import tpu_sc as plsc

N = 50000
E = 800000
H = 128
NC, NS, LANES = 2, 16, 16
NW = NC * NS
EPAD = 819200
ER = EPAD // 128
TPW = EPAD // NW
MBR = TPW // 128 // 8
ACC_ROWS = 51200
SLICES = 8
SW = H // SLICES
STRIPE = ACC_ROWS // NS
NB = 2000
GRID = N // NB

_f32 = jnp.float32
_i32 = jnp.int32


def _elu(v):
    return jnp.where(v > 0.0, v, jnp.exp(v) - 1.0)


def _enc_body(x_ref, wn_ref, bn_ref, wm_ref, bm_ref, h_ref, *outs):
    h = _elu(jnp.dot(x_ref[...], wn_ref[...], preferred_element_type=_f32)
             + bn_ref[...])
    h_ref[...] = h
    hw = jnp.dot(h, wm_ref[...], preferred_element_type=_f32) + bm_ref[...]
    for k in range(SLICES):
        outs[k][...] = hw[:, SW * k:SW * (k + 1)]


def _encode(x, wn, bn, wm, bm):
    return pl.pallas_call(
        _enc_body,
        grid=(GRID,),
        in_specs=[
            pl.BlockSpec((NB, 4), lambda i: (i, 0)),
            pl.BlockSpec((4, H), lambda i: (0, 0)),
            pl.BlockSpec((1, H), lambda i: (0, 0)),
            pl.BlockSpec((H, H), lambda i: (0, 0)),
            pl.BlockSpec((1, H), lambda i: (0, 0)),
        ],
        out_specs=[pl.BlockSpec((NB, H), lambda i: (i, 0))]
        + [pl.BlockSpec((NB, SW), lambda i: (i, 0)) for _ in range(SLICES)],
        out_shape=[jax.ShapeDtypeStruct((N, H), _f32)]
        + [jax.ShapeDtypeStruct((N, SW), _f32) for _ in range(SLICES)],
    )(x, wn, bn, wm, bm)


def _acc_update(h_ref, agg_ref, wu_ref, bu_ref):
    a = jnp.concatenate(
        [agg_ref[0, k] + agg_ref[1, k] for k in range(SLICES)], axis=-1)
    acc = jnp.dot(a, wu_ref[...], preferred_element_type=_f32) + bu_ref[...]
    return _elu(h_ref[...] + acc)


def _upd_body(h_ref, agg_ref, wu_ref, bu_ref, wm_ref, bm_ref,
              h_out, *outs):
    h = _acc_update(h_ref, agg_ref, wu_ref, bu_ref)
    h_out[...] = h
    hw = jnp.dot(h, wm_ref[...], preferred_element_type=_f32) + bm_ref[...]
    for k in range(SLICES):
        outs[k][...] = hw[:, SW * k:SW * (k + 1)]


def _update(h, agg, wuS, bu, wm, bm):
    return pl.pallas_call(
        _upd_body,
        grid=(GRID,),
        in_specs=[
            pl.BlockSpec((NB, H), lambda i: (i, 0)),
            pl.BlockSpec((2, SLICES, NB, SW), lambda i: (0, 0, i, 0)),
            pl.BlockSpec((H, H), lambda i: (0, 0)),
            pl.BlockSpec((1, H), lambda i: (0, 0)),
            pl.BlockSpec((H, H), lambda i: (0, 0)),
            pl.BlockSpec((1, H), lambda i: (0, 0)),
        ],
        out_specs=[pl.BlockSpec((NB, H), lambda i: (i, 0))]
        + [pl.BlockSpec((NB, SW), lambda i: (i, 0)) for _ in range(SLICES)],
        out_shape=[jax.ShapeDtypeStruct((N, H), _f32)]
        + [jax.ShapeDtypeStruct((N, SW), _f32) for _ in range(SLICES)],
    )(h, agg, wuS, bu, wm, bm)


def _fin_body(h_ref, agg_ref, wu_ref, bu_ref, wt1_ref, bt1_ref, wt2_ref,
              bt2_ref, h_out, topo_out):
    h = _acc_update(h_ref, agg_ref, wu_ref, bu_ref)
    h_out[...] = h
    t1 = _elu(jnp.dot(h, wt1_ref[...], preferred_element_type=_f32)
              + bt1_ref[...])
    topo_out[...] = (jnp.dot(t1, wt2_ref[...], preferred_element_type=_f32)
                     + bt2_ref[...])


def _final(h, agg, wuS, bu, wt1, bt1, wt2, bt2):
    return pl.pallas_call(
        _fin_body,
        grid=(GRID,),
        in_specs=[
            pl.BlockSpec((NB, H), lambda i: (i, 0)),
            pl.BlockSpec((2, SLICES, NB, SW), lambda i: (0, 0, i, 0)),
            pl.BlockSpec((H, H), lambda i: (0, 0)),
            pl.BlockSpec((1, H), lambda i: (0, 0)),
            pl.BlockSpec((H, 64), lambda i: (0, 0)),
            pl.BlockSpec((1, 64), lambda i: (0, 0)),
            pl.BlockSpec((64, 1), lambda i: (0, 0)),
            pl.BlockSpec((1, 1), lambda i: (0, 0)),
        ],
        out_specs=[
            pl.BlockSpec((NB, H), lambda i: (i, 0)),
            pl.BlockSpec((NB, 1), lambda i: (i, 0)),
        ],
        out_shape=[
            jax.ShapeDtypeStruct((N, H), _f32),
            jax.ShapeDtypeStruct((N, 1), _f32),
        ],
    )(h, agg, wuS, bu, wt1, bt1, wt2, bt2)


def _sc_compute(rows, attr_v, w0, b0, ph, j):
    def group_step(g, carry):
        av = attr_v[ph, j, pl.ds(g * LANES, LANES)]
        for t in range(LANES):
            e = g * LANES + t
            a = av[t]
            e0 = _elu(a * w0 + b0)
            r0 = rows[ph, j, e, pl.ds(0, SW)]
            rows[ph, j, e, pl.ds(0, SW)] = _elu(r0 + e0)
        return carry

    lax.fori_loop(0, 8, group_step, 0)


def _sc_edge_body(*refs):
    ts = refs[:SLICES]
    (src_r, dst_r, attr_r, we_r, be_r, out,
     src_v, dst_v, attr_v, rows, wev, bev, zbuf, acc, isem, gsem, ssem) = \
        refs[SLICES:]
    c = lax.axis_index("c")
    s = lax.axis_index("s")
    wid = c * NS + s
    tbase = wid * (MBR * 8)

    pltpu.sync_copy(we_r, wev)
    pltpu.sync_copy(be_r, bev)

    def zb(r, carry):
        zbuf[r, pl.ds(0, SW)] = jnp.zeros((SW,), _f32)
        return carry

    lax.fori_loop(0, 200, zb, 0)

    def load_idx_sync(it, ph):
        pltpu.sync_copy(src_r.at[pl.ds(tbase + it * 8, 8)], src_v.at[ph])
        pltpu.sync_copy(dst_r.at[pl.ds(tbase + it * 8, 8)], dst_v.at[ph])
        pltpu.sync_copy(attr_r.at[pl.ds(tbase + it * 8, 8)], attr_v.at[ph])

    def fire_idx(it, ph):
        pltpu.async_copy(src_r.at[pl.ds(tbase + it * 8, 8)], src_v.at[ph],
                         isem.at[ph])
        pltpu.async_copy(dst_r.at[pl.ds(tbase + it * 8, 8)], dst_v.at[ph],
                         isem.at[ph])
        pltpu.async_copy(attr_r.at[pl.ds(tbase + it * 8, 8)], attr_v.at[ph],
                         isem.at[ph])

    def wait_idx(it, ph):
        pltpu.make_async_copy(src_r.at[pl.ds(tbase + it * 8, 8)],
                              src_v.at[ph], isem.at[ph]).wait()
        pltpu.make_async_copy(dst_r.at[pl.ds(tbase + it * 8, 8)],
                              dst_v.at[ph], isem.at[ph]).wait()
        pltpu.make_async_copy(attr_r.at[pl.ds(tbase + it * 8, 8)],
                              attr_v.at[ph], isem.at[ph]).wait()

    for k in range(SLICES):
        tk = ts[k]
        w0 = wev[pl.ds(SW * k, SW)]
        b0 = bev[pl.ds(SW * k, SW)]

        def fire_gathers(ph, carry=0):
            def fg(j, carry2):
                pltpu.async_copy(tk.at[src_v.at[ph, j]], rows.at[ph, j],
                                 gsem.at[ph])
                return carry2
            lax.fori_loop(0, 8, fg, 0)
            return carry

        def wait_gathers(ph):
            def wg(j, carry2):
                pltpu.make_async_copy(tk.at[src_v.at[ph, j]],
                                      rows.at[ph, j], gsem.at[ph]).wait()
                return carry2
            lax.fori_loop(0, 8, wg, 0)

        def fire_scatters(ph):
            def fs(j, carry2):
                pltpu.async_copy(rows.at[ph, j], acc.at[dst_v.at[ph, j]],
                                 ssem.at[ph], add=True)
                return carry2
            lax.fori_loop(0, 8, fs, 0)

        def drain_scatters(ph):
            def dsc(j, carry2):
                pltpu.make_async_copy(rows.at[ph, j],
                                      acc.at[dst_v.at[ph, j]],
                                      ssem.at[ph]).wait()
                return carry2
            lax.fori_loop(0, 8, dsc, 0)

        def zstripe(r, carry):
            pltpu.sync_copy(zbuf, acc.at[pl.ds((s * 16 + r) * 200, 200)])
            return carry

        lax.fori_loop(0, 16, zstripe, 0)
        plsc.subcore_barrier()

        load_idx_sync(0, 0)
        fire_gathers(0)
        fire_idx(1, 1)

        def macro(it, carry):
            p = lax.rem(it, 3)
            pn = lax.rem(it + 1, 3)
            pnn = lax.rem(it + 2, 3)

            @pl.when(it + 1 < MBR)
            def _():
                wait_idx(it + 1, pn)
                fire_gathers(pn)

            @pl.when(it >= 1)
            def _():
                drain_scatters(pnn)

            @pl.when(it + 2 < MBR)
            def _():
                fire_idx(it + 2, pnn)

            wait_gathers(p)

            def cstep(j, carry2):
                _sc_compute(rows, attr_v, w0, b0, p, j)
                return carry2

            lax.fori_loop(0, 8, cstep, 0)
            fire_scatters(p)
            return carry

        lax.fori_loop(0, MBR, macro, 0)
        drain_scatters((MBR - 1) % 3)
        plsc.subcore_barrier()
        pltpu.sync_copy(acc.at[pl.ds(s * STRIPE, STRIPE)],
                        out.at[c, k, pl.ds(s * STRIPE, STRIPE)])


_edge_sc = functools.partial(
    pl.kernel,
    out_type=jax.ShapeDtypeStruct((2, SLICES, ACC_ROWS, SW), _f32),
    mesh=plsc.VectorSubcoreMesh(core_axis_name="c", subcore_axis_name="s"),
    compiler_params=pltpu.CompilerParams(use_tc_tiling_on_sc=False),
    scratch_types=[
        pltpu.VMEM((3, 8, 128), _i32),
        pltpu.VMEM((3, 8, 128), _i32),
        pltpu.VMEM((3, 8, 128), _f32),
        pltpu.VMEM((3, 8, 128, SW), _f32),
        pltpu.VMEM((H,), _f32),
        pltpu.VMEM((H,), _f32),
        pltpu.VMEM((200, SW), _f32),
        pltpu.VMEM_SHARED((ACC_ROWS, SW), _f32),
        pltpu.SemaphoreType.DMA((3,)),
        pltpu.SemaphoreType.DMA((3,)),
        pltpu.SemaphoreType.DMA((3,)),
    ],
)(_sc_edge_body)


def _rows_body(uvw_ref, h_ref, out_ref):
    i = pl.program_id(0)
    r = uvw_ref[i] % 8
    out_ref[pl.ds(i, 1), :] = h_ref[pl.ds(r, 1), :]


def _head_rows(uvw, h):
    return pl.pallas_call(
        _rows_body,
        grid_spec=pltpu.PrefetchScalarGridSpec(
            num_scalar_prefetch=1,
            grid=(3,),
            in_specs=[pl.BlockSpec((8, H), lambda i, uvw: (uvw[i] // 8, 0))],
            out_specs=pl.BlockSpec((3, H), lambda i, uvw: (0, 0)),
        ),
        out_shape=jax.ShapeDtypeStruct((3, H), _f32),
    )(uvw, h)


def _head_body(r3, zc, tcrd, w1t, w1f, w1z, b1, w2, b2, wmu, bmu, wlv, blv,
               wd1z, wd1f, wd1zc, bd1, wd2, bd2, wd3, bd3, wp1f, wp1z, bp1,
               wp2, bp2, pbs, xp_ref, mu_ref, lv_ref):
    def dot(a, b):
        return jnp.dot(a, b, preferred_element_type=_f32)

    feat = (r3[0:1, :] + r3[1:2, :] + r3[2:3, :]) * (1.0 / 3.0)
    he = _elu(dot(tcrd[...], w1t[...]) + dot(feat, w1f[...])
              + dot(zc[...], w1z[...]) + b1[...])
    he = _elu(dot(he, w2[...]) + b2[...])
    mu = dot(he, wmu[...]) + bmu[...]
    lv = dot(he, wlv[...]) + blv[...]
    mu_ref[...] = mu
    lv_ref[...] = lv
    hd = _elu(dot(mu, wd1z[...]) + dot(feat, wd1f[...])
              + dot(zc[...], wd1zc[...]) + bd1[...])
    hd = _elu(dot(hd, wd2[...]) + bd2[...])
    base = dot(hd, wd3[...]) + bd3[...]
    pb = jnp.tanh(dot(_elu(dot(feat, wp1f[...]) + dot(zc[...], wp1z[...])
                           + bp1[...]), wp2[...]) + bp2[...])
    scale = jnp.clip(pbs[...], 0.0, 0.5)
    xp_ref[...] = base + pb * scale


def _head(r3, zc, tcrd, args):
    ins = [r3, zc, tcrd] + list(args)
    return pl.pallas_call(
        _head_body,
        in_specs=[pl.BlockSpec(a.shape, lambda: (0, 0)) for a in ins],
        out_specs=[
            pl.BlockSpec((1, 4), lambda: (0, 0)),
            pl.BlockSpec((1, 64), lambda: (0, 0)),
            pl.BlockSpec((1, 64), lambda: (0, 0)),
        ],
        out_shape=[
            jax.ShapeDtypeStruct((1, 4), _f32),
            jax.ShapeDtypeStruct((1, 64), _f32),
            jax.ShapeDtypeStruct((1, 64), _f32),
        ],
    )(*ins)


def kernel(x, edge_attr, z_c, true_coords, params, edge_index, u_idx, v_idx,
           w_idx):
    p = params
    src = edge_index[0].astype(_i32)
    dst = edge_index[1].astype(_i32)
    pad = EPAD - E
    srcp = jnp.concatenate([src, jnp.zeros((pad,), _i32)]).reshape(ER, 128)
    dump = N + jnp.arange(pad, dtype=_i32) % (ACC_ROWS - N)
    dstp = jnp.concatenate([dst, dump]).reshape(ER, 128)
    attrp = jnp.concatenate(
        [edge_attr[:, 0].astype(_f32), jnp.zeros((pad,), _f32)]
    ).reshape(ER, 128)
    we_r = p['We'][0]
    be_r = p['be']

    h, *ts = _encode(
        x, p['Wn'], p['bn'].reshape(1, H), p['Wm0'], p['bm0'].reshape(1, H))

    for l in range(4):
        agg = _edge_sc(*ts, srcp, dstp, attrp, we_r, be_r)
        wuS = p['Wu%d' % l]
        bu = p['bu%d' % l].reshape(1, H)
        if l < 3:
            h, *ts = _update(
                h, agg, wuS, bu,
                p['Wm%d' % (l + 1)], p['bm%d' % (l + 1)].reshape(1, H))
        else:
            h, topo2 = _final(
                h, agg, wuS, bu,
                p['Wt1'], p['bt1'].reshape(1, 64),
                p['Wt2'], p['bt2'].reshape(1, 1))

    uvw = jnp.stack([jnp.asarray(u_idx, _i32), jnp.asarray(v_idx, _i32),
                     jnp.asarray(w_idx, _i32)])
    r3 = _head_rows(uvw, h)

    we1 = p['We1']
    wd1 = p['Wd1']
    wp1 = p['Wp1']
    head_args = (
        we1[0:4], we1[4:132], we1[132:260], p['be1'].reshape(1, H),
        p['We2'], p['be2'].reshape(1, 64),
        p['Wmu'], p['bmu'].reshape(1, 64),
        p['Wlv'], p['blv'].reshape(1, 64),
        wd1[0:64], wd1[64:192], wd1[192:320], p['bd1'].reshape(1, H),
        p['Wd2'], p['bd2'].reshape(1, 64),
        p['Wd3'], p['bd3'].reshape(1, 4),
        wp1[0:128], wp1[128:256], p['bp1'].reshape(1, 64),
        p['Wp2'], p['bp2'].reshape(1, 4),
        p['pbs'].reshape(1, 1),
    )
    xp, mu, lv = _head(r3, z_c, true_coords, head_args)
    return topo2[:, 0], xp, mu, lv
